# Initial kernel scaffold; baseline (speedup 1.0000x reference)
import jax
import jax.numpy as jnp
from jax.experimental import pallas as pl


def kernel(x, x_src, edge_index, cg_edge_index, edge_attr, u, batch, batch_src, Wq, Wk, We1, be1, We2, be2, Wn1, bn1, Wn2, bn2, Wg1, bg1, Wg2, bg2):
    raise NotImplementedError("write your pallas kernel here")


# speedup vs baseline: 1.2760x; 1.2760x over previous
import functools

import jax
import jax.numpy as jnp
from jax import lax
from jax.experimental import pallas as pl
from jax.experimental.pallas import tpu as pltpu

N = 10000
NS = 10000
E = 160000
ECG = 160000
D = 256
DE = 16
DEO = 128
DU = 64
DA = 64
B = 4

BM = 2000
BE = 2000


def _proj_body(x_ref, xs_ref, wcat_ref, wk_ref, xcat_ref, k_ref):
    xcat_ref[...] = jnp.dot(x_ref[...], wcat_ref[...],
                            preferred_element_type=jnp.float32)
    k_ref[...] = jnp.dot(xs_ref[...], wk_ref[...],
                         preferred_element_type=jnp.float32)


def _proj(x, x_src, Wcat, Wk):
    DC = Wcat.shape[1]
    return pl.pallas_call(
        _proj_body,
        grid=(N // BM,),
        in_specs=[
            pl.BlockSpec((BM, D), lambda i: (i, 0)),
            pl.BlockSpec((BM, D), lambda i: (i, 0)),
            pl.BlockSpec((D, DC), lambda i: (0, 0)),
            pl.BlockSpec((D, DA), lambda i: (0, 0)),
        ],
        out_specs=[
            pl.BlockSpec((BM, DC), lambda i: (i, 0)),
            pl.BlockSpec((BM, DA), lambda i: (i, 0)),
        ],
        out_shape=[
            jax.ShapeDtypeStruct((N, DC), jnp.float32),
            jax.ShapeDtypeStruct((NS, DA), jnp.float32),
        ],
    )(x, x_src, Wcat, Wk)


def _tables_body(a_ref, xr_ref, xc_ref, xn_ref, un_ref, un2_ref,
                 pr_ref, pc_ref, pn_ref):
    av = a_ref[...]
    pr_ref[...] = av * xr_ref[...] + un_ref[...]
    pc_ref[...] = av * xc_ref[...]
    pn_ref[...] = av * xn_ref[...] + un2_ref[...]


def _tables(A, Xr, Xc, Xn, Un, Un2):
    spec = pl.BlockSpec((BM, D), lambda i: (i, 0))
    return pl.pallas_call(
        _tables_body,
        grid=(N // BM,),
        in_specs=[spec] * 6,
        out_specs=[spec] * 3,
        out_shape=[jax.ShapeDtypeStruct((N, D), jnp.float32)] * 3,
    )(A, Xr, Xc, Xn, Un, Un2)


def _edge_body(g_ref, ea_ref, we1e_ref, we2_ref, be2_ref, out_ref):
    h = g_ref[...] + jnp.dot(ea_ref[...], we1e_ref[...],
                             preferred_element_type=jnp.float32)
    h = jnp.maximum(h, 0.0)
    out_ref[...] = jnp.dot(h, we2_ref[...],
                           preferred_element_type=jnp.float32) + be2_ref[...]


def _edge_mlp(G, edge_attr, We1e, We2, be2row):
    return pl.pallas_call(
        _edge_body,
        grid=(E // BE,),
        in_specs=[
            pl.BlockSpec((BE, D), lambda i: (i, 0)),
            pl.BlockSpec((BE, DE), lambda i: (i, 0)),
            pl.BlockSpec((DE, D), lambda i: (0, 0)),
            pl.BlockSpec((D, DEO), lambda i: (0, 0)),
            pl.BlockSpec((1, DEO), lambda i: (0, 0)),
        ],
        out_specs=pl.BlockSpec((BE, DEO), lambda i: (i, 0)),
        out_shape=jax.ShapeDtypeStruct((E, DEO), jnp.float32),
    )(G, edge_attr, We1e, We2, be2row)


def _node_body(pn_ref, aggm_ref, wn1a_ref, wn2_ref, bn2_ref, bh_ref,
               u_ref, wg1p_ref, wg1u_ref, bg1_ref, wg2_ref, bg2_ref,
               pcinv_ref, xout_ref, uout_ref, psum_scr):
    i = pl.program_id(0)
    npre = pn_ref[...] + jnp.dot(aggm_ref[...], wn1a_ref[...],
                                 preferred_element_type=jnp.float32)
    h = jnp.maximum(npre, 0.0)
    xout = jnp.dot(h, wn2_ref[...],
                   preferred_element_type=jnp.float32) + bn2_ref[...]
    xout_ref[...] = xout

    part = lax.dot_general(bh_ref[...], xout, (((0,), (0,)), ((), ())),
                           preferred_element_type=jnp.float32)

    @pl.when(i == 0)
    def _init():
        psum_scr[...] = jnp.zeros_like(psum_scr)

    psum_scr[:B, :] += part

    @pl.when(i == pl.num_programs(0) - 1)
    def _fin():
        pool = psum_scr[:B, :] * pcinv_ref[...]
        h2 = jnp.dot(pool, wg1p_ref[...], preferred_element_type=jnp.float32)
        h2 = h2 + jnp.dot(u_ref[...], wg1u_ref[...],
                          preferred_element_type=jnp.float32) + bg1_ref[...]
        h2 = jnp.maximum(h2, 0.0)
        uout_ref[...] = jnp.dot(h2, wg2_ref[...],
                                preferred_element_type=jnp.float32) + bg2_ref[...]


def _node_global(Pn, AggM, Wn1a, Wn2, bn2row, Bh, u, Wg1p, Wg1u, bg1row,
                 Wg2, bg2row, PcInv):
    return pl.pallas_call(
        _node_body,
        grid=(N // BM,),
        in_specs=[
            pl.BlockSpec((BM, D), lambda i: (i, 0)),
            pl.BlockSpec((BM, DEO), lambda i: (i, 0)),
            pl.BlockSpec((DEO, D), lambda i: (0, 0)),
            pl.BlockSpec((D, D), lambda i: (0, 0)),
            pl.BlockSpec((1, D), lambda i: (0, 0)),
            pl.BlockSpec((BM, B), lambda i: (i, 0)),
            pl.BlockSpec((B, DU), lambda i: (0, 0)),
            pl.BlockSpec((D, 128), lambda i: (0, 0)),
            pl.BlockSpec((DU, 128), lambda i: (0, 0)),
            pl.BlockSpec((1, 128), lambda i: (0, 0)),
            pl.BlockSpec((128, DU), lambda i: (0, 0)),
            pl.BlockSpec((1, DU), lambda i: (0, 0)),
            pl.BlockSpec((B, D), lambda i: (0, 0)),
        ],
        out_specs=[
            pl.BlockSpec((BM, D), lambda i: (i, 0)),
            pl.BlockSpec((B, DU), lambda i: (0, 0)),
        ],
        out_shape=[
            jax.ShapeDtypeStruct((N, D), jnp.float32),
            jax.ShapeDtypeStruct((B, DU), jnp.float32),
        ],
        scratch_shapes=[pltpu.VMEM((8, D), jnp.float32)],
    )(Pn, AggM, Wn1a, Wn2, bn2row, Bh, u, Wg1p, Wg1u, bg1row, Wg2, bg2row,
      PcInv)


def _attention_sums(q, k, ci, cj):
    scores = jax.nn.sigmoid(jnp.sum(q[ci] * k[cj], axis=-1) / 8.0)
    ssum = jax.ops.segment_sum(scores, ci, num_segments=N)
    cnt = jax.ops.segment_sum(jnp.ones_like(scores), ci, num_segments=N)
    return ssum, cnt


def _gather_sum(Pr, Pc, row, col):
    return Pr[row] + Pc[col]


def _scatter_mean(eout, row):
    agg = jax.ops.segment_sum(eout, row, num_segments=N)
    ecnt = jax.ops.segment_sum(jnp.ones((E,), jnp.float32), row,
                               num_segments=N)
    return agg / jnp.maximum(ecnt, 1.0)[:, None]


def kernel(x, x_src, edge_index, cg_edge_index, edge_attr, u, batch,
           batch_src, Wq, Wk, We1, be1, We2, be2, Wn1, bn1, Wn2, bn2,
           Wg1, bg1, Wg2, bg2):
    We1_r = We1[:D]
    We1_c = We1[D:2 * D]
    We1_e = We1[2 * D:2 * D + DE]
    We1_u = We1[2 * D + DE:]
    Wn1_x = Wn1[:D]
    Wn1_a = Wn1[D:D + DEO]
    Wn1_u = Wn1[D + DEO:]

    Wcat = jnp.concatenate([Wq, We1_r, We1_c, Wn1_x], axis=1)
    Xcat, k = _proj(x, x_src, Wcat, Wk)
    q = Xcat[:, :DA]
    Xr = Xcat[:, DA:DA + D]
    Xc = Xcat[:, DA + D:DA + 2 * D]
    Xn = Xcat[:, DA + 2 * D:]

    ci = cg_edge_index[0]
    cj = cg_edge_index[1]
    ssum, cnt = _attention_sums(q, k, ci, cj)
    a = ssum / jnp.maximum(cnt, 1.0)

    A = jnp.broadcast_to(a[:, None], (N, D))
    Un = (u @ We1_u + be1)[batch]
    Un2 = (u @ Wn1_u + bn1)[batch]
    Pr, Pc, Pn = _tables(A, Xr, Xc, Xn, Un, Un2)

    row = edge_index[0]
    col = edge_index[1]
    G = _gather_sum(Pr, Pc, row, col)

    eout = _edge_mlp(G, edge_attr, We1_e, We2, be2.reshape(1, DEO))

    AggM = _scatter_mean(eout, row)

    Bh = jax.nn.one_hot(batch, B, dtype=jnp.float32)
    pcnt = jnp.maximum(jnp.sum(Bh, axis=0), 1.0)
    PcInv = jnp.broadcast_to((1.0 / pcnt)[:, None], (B, D))
    x_out, u_out = _node_global(Pn, AggM, Wn1_a, Wn2, bn2.reshape(1, D),
                                Bh, u, Wg1[:D], Wg1[D:], bg1.reshape(1, 128),
                                Wg2, bg2.reshape(1, DU), PcInv)
    return (x_out, eout, u_out)


# speedup vs baseline: 1.7878x; 1.4011x over previous
import functools

import jax
import jax.numpy as jnp
from jax import lax
from jax.experimental import pallas as pl
from jax.experimental.pallas import tpu as pltpu
from jax.experimental.pallas import tpu_sc as plsc

N = 10000
NS = 10000
E = 160000
ECG = 160000
D = 256
DE = 16
DEO = 128
DU = 64
DA = 64
B = 4

BM = 2000
BE = 2000


def _proj_body(x_ref, xs_ref, wcat_ref, wk_ref, xcat_ref, k_ref):
    xcat_ref[...] = jnp.dot(x_ref[...], wcat_ref[...],
                            preferred_element_type=jnp.float32)
    k_ref[...] = jnp.dot(xs_ref[...], wk_ref[...],
                         preferred_element_type=jnp.float32)


def _proj(x, x_src, Wcat, Wk):
    DC = Wcat.shape[1]
    return pl.pallas_call(
        _proj_body,
        grid=(N // BM,),
        in_specs=[
            pl.BlockSpec((BM, D), lambda i: (i, 0)),
            pl.BlockSpec((BM, D), lambda i: (i, 0)),
            pl.BlockSpec((D, DC), lambda i: (0, 0)),
            pl.BlockSpec((D, DA), lambda i: (0, 0)),
        ],
        out_specs=[
            pl.BlockSpec((BM, DC), lambda i: (i, 0)),
            pl.BlockSpec((BM, DA), lambda i: (i, 0)),
        ],
        out_shape=[
            jax.ShapeDtypeStruct((N, DC), jnp.float32),
            jax.ShapeDtypeStruct((NS, DA), jnp.float32),
        ],
    )(x, x_src, Wcat, Wk)


def _tables_body(a_ref, xr_ref, xc_ref, xn_ref, un_ref, un2_ref,
                 pr_ref, pc_ref, pn_ref):
    av = a_ref[...]
    pr_ref[...] = av * xr_ref[...] + un_ref[...]
    pc_ref[...] = av * xc_ref[...]
    pn_ref[...] = av * xn_ref[...] + un2_ref[...]


def _tables(A, Xr, Xc, Xn, Un, Un2):
    spec = pl.BlockSpec((BM, D), lambda i: (i, 0))
    return pl.pallas_call(
        _tables_body,
        grid=(N // BM,),
        in_specs=[spec] * 6,
        out_specs=[spec] * 3,
        out_shape=[jax.ShapeDtypeStruct((N, D), jnp.float32)] * 3,
    )(A, Xr, Xc, Xn, Un, Un2)


def _edge_body(g_ref, ea_ref, we1e_ref, we2_ref, be2_ref, out_ref):
    h = g_ref[...] + jnp.dot(ea_ref[...], we1e_ref[...],
                             preferred_element_type=jnp.float32)
    h = jnp.maximum(h, 0.0)
    out_ref[...] = jnp.dot(h, we2_ref[...],
                           preferred_element_type=jnp.float32) + be2_ref[...]


def _edge_mlp(G, edge_attr, We1e, We2, be2row):
    return pl.pallas_call(
        _edge_body,
        grid=(E // BE,),
        in_specs=[
            pl.BlockSpec((BE, D), lambda i: (i, 0)),
            pl.BlockSpec((BE, DE), lambda i: (i, 0)),
            pl.BlockSpec((DE, D), lambda i: (0, 0)),
            pl.BlockSpec((D, DEO), lambda i: (0, 0)),
            pl.BlockSpec((1, DEO), lambda i: (0, 0)),
        ],
        out_specs=pl.BlockSpec((BE, DEO), lambda i: (i, 0)),
        out_shape=jax.ShapeDtypeStruct((E, DEO), jnp.float32),
    )(G, edge_attr, We1e, We2, be2row)


def _node_body(pn_ref, aggm_ref, wn1a_ref, wn2_ref, bn2_ref, bh_ref,
               u_ref, wg1p_ref, wg1u_ref, bg1_ref, wg2_ref, bg2_ref,
               pcinv_ref, xout_ref, uout_ref, psum_scr):
    i = pl.program_id(0)
    npre = pn_ref[...] + jnp.dot(aggm_ref[...], wn1a_ref[...],
                                 preferred_element_type=jnp.float32)
    h = jnp.maximum(npre, 0.0)
    xout = jnp.dot(h, wn2_ref[...],
                   preferred_element_type=jnp.float32) + bn2_ref[...]
    xout_ref[...] = xout

    part = lax.dot_general(bh_ref[...], xout, (((0,), (0,)), ((), ())),
                           preferred_element_type=jnp.float32)

    @pl.when(i == 0)
    def _init():
        psum_scr[...] = jnp.zeros_like(psum_scr)

    psum_scr[:B, :] += part

    @pl.when(i == pl.num_programs(0) - 1)
    def _fin():
        pool = psum_scr[:B, :] * pcinv_ref[...]
        h2 = jnp.dot(pool, wg1p_ref[...], preferred_element_type=jnp.float32)
        h2 = h2 + jnp.dot(u_ref[...], wg1u_ref[...],
                          preferred_element_type=jnp.float32) + bg1_ref[...]
        h2 = jnp.maximum(h2, 0.0)
        uout_ref[...] = jnp.dot(h2, wg2_ref[...],
                                preferred_element_type=jnp.float32) + bg2_ref[...]


def _node_global(Pn, AggM, Wn1a, Wn2, bn2row, Bh, u, Wg1p, Wg1u, bg1row,
                 Wg2, bg2row, PcInv):
    return pl.pallas_call(
        _node_body,
        grid=(N // BM,),
        in_specs=[
            pl.BlockSpec((BM, D), lambda i: (i, 0)),
            pl.BlockSpec((BM, DEO), lambda i: (i, 0)),
            pl.BlockSpec((DEO, D), lambda i: (0, 0)),
            pl.BlockSpec((D, D), lambda i: (0, 0)),
            pl.BlockSpec((1, D), lambda i: (0, 0)),
            pl.BlockSpec((BM, B), lambda i: (i, 0)),
            pl.BlockSpec((B, DU), lambda i: (0, 0)),
            pl.BlockSpec((D, 128), lambda i: (0, 0)),
            pl.BlockSpec((DU, 128), lambda i: (0, 0)),
            pl.BlockSpec((1, 128), lambda i: (0, 0)),
            pl.BlockSpec((128, DU), lambda i: (0, 0)),
            pl.BlockSpec((1, DU), lambda i: (0, 0)),
            pl.BlockSpec((B, D), lambda i: (0, 0)),
        ],
        out_specs=[
            pl.BlockSpec((BM, D), lambda i: (i, 0)),
            pl.BlockSpec((B, DU), lambda i: (0, 0)),
        ],
        out_shape=[
            jax.ShapeDtypeStruct((N, D), jnp.float32),
            jax.ShapeDtypeStruct((B, DU), jnp.float32),
        ],
        scratch_shapes=[pltpu.VMEM((8, D), jnp.float32)],
    )(Pn, AggM, Wn1a, Wn2, bn2row, Bh, u, Wg1p, Wg1u, bg1row, Wg2, bg2row,
      PcInv)


NC = 2
NSUB = 16
NW = NC * NSUB
CG = 128
NCHUNK = E // CG


def _sc_mesh():
    return plsc.VectorSubcoreMesh(core_axis_name="c", subcore_axis_name="s",
                                  num_cores=NC, num_subcores=NSUB)


def _worker_id():
    return lax.axis_index("s") * NC + lax.axis_index("c")


def _attention_sums(q, k, ci, cj):
    scores = jax.nn.sigmoid(jnp.sum(q[ci] * k[cj], axis=-1) / 8.0)
    ssum = jax.ops.segment_sum(scores, ci, num_segments=N)
    cnt = jax.ops.segment_sum(jnp.ones_like(scores), ci, num_segments=N)
    return ssum, cnt


def _gather_body(pr_hbm, pc_hbm, row_hbm, col_hbm, g_hbm,
                 ridx, cidx, rbuf, cbuf, sem1, sem2):
    w = _worker_id()
    nchunks = (NCHUNK - w + NW - 1) // NW

    def chunk(t, _):
        off = (t * NW + w) * CG
        pltpu.sync_copy(row_hbm.at[pl.ds(off, CG)], ridx)
        pltpu.sync_copy(col_hbm.at[pl.ds(off, CG)], cidx)
        cp1 = pltpu.async_copy(pr_hbm.at[ridx], rbuf, sem1)
        cp2 = pltpu.async_copy(pc_hbm.at[cidx], cbuf, sem2)
        cp1.wait()
        cp2.wait()

        def add_row(r, _):
            for j in range(D // 16):
                sl = pl.ds(j * 16, 16)
                rbuf[r, sl] = rbuf[r, sl] + cbuf[r, sl]
            return 0

        lax.fori_loop(0, CG, add_row, 0)
        pltpu.sync_copy(rbuf, g_hbm.at[pl.ds(off, CG)])
        return 0

    lax.fori_loop(0, nchunks, chunk, 0)


def _gather_sum(Pr, Pc, row, col):
    call = pl.kernel(
        _gather_body,
        out_type=jax.ShapeDtypeStruct((E, D), jnp.float32),
        mesh=_sc_mesh(),
        scratch_types=[
            pltpu.VMEM((CG,), jnp.int32),
            pltpu.VMEM((CG,), jnp.int32),
            pltpu.VMEM((CG, D), jnp.float32),
            pltpu.VMEM((CG, D), jnp.float32),
            pltpu.SemaphoreType.DMA,
            pltpu.SemaphoreType.DMA,
        ],
    )
    return call(Pr, Pc, row, col)


def _scatter_mean(eout, row):
    agg = jax.ops.segment_sum(eout, row, num_segments=N)
    ecnt = jax.ops.segment_sum(jnp.ones((E,), jnp.float32), row,
                               num_segments=N)
    return agg / jnp.maximum(ecnt, 1.0)[:, None]


def kernel(x, x_src, edge_index, cg_edge_index, edge_attr, u, batch,
           batch_src, Wq, Wk, We1, be1, We2, be2, Wn1, bn1, Wn2, bn2,
           Wg1, bg1, Wg2, bg2):
    We1_r = We1[:D]
    We1_c = We1[D:2 * D]
    We1_e = We1[2 * D:2 * D + DE]
    We1_u = We1[2 * D + DE:]
    Wn1_x = Wn1[:D]
    Wn1_a = Wn1[D:D + DEO]
    Wn1_u = Wn1[D + DEO:]

    Wcat = jnp.concatenate([Wq, We1_r, We1_c, Wn1_x], axis=1)
    Xcat, k = _proj(x, x_src, Wcat, Wk)
    q = Xcat[:, :DA]
    Xr = Xcat[:, DA:DA + D]
    Xc = Xcat[:, DA + D:DA + 2 * D]
    Xn = Xcat[:, DA + 2 * D:]

    ci = cg_edge_index[0]
    cj = cg_edge_index[1]
    ssum, cnt = _attention_sums(q, k, ci, cj)
    a = ssum / jnp.maximum(cnt, 1.0)

    A = jnp.broadcast_to(a[:, None], (N, D))
    Un = (u @ We1_u + be1)[batch]
    Un2 = (u @ Wn1_u + bn1)[batch]
    Pr, Pc, Pn = _tables(A, Xr, Xc, Xn, Un, Un2)

    row = edge_index[0]
    col = edge_index[1]
    G = _gather_sum(Pr, Pc, row, col)

    eout = _edge_mlp(G, edge_attr, We1_e, We2, be2.reshape(1, DEO))

    AggM = _scatter_mean(eout, row)

    Bh = jax.nn.one_hot(batch, B, dtype=jnp.float32)
    pcnt = jnp.maximum(jnp.sum(Bh, axis=0), 1.0)
    PcInv = jnp.broadcast_to((1.0 / pcnt)[:, None], (B, D))
    x_out, u_out = _node_global(Pn, AggM, Wn1_a, Wn2, bn2.reshape(1, D),
                                Bh, u, Wg1[:D], Wg1[D:], bg1.reshape(1, 128),
                                Wg2, bg2.reshape(1, DU), PcInv)
    return (x_out, eout, u_out)


# speedup vs baseline: 6.0502x; 3.3841x over previous
import functools

import jax
import jax.numpy as jnp
from jax import lax
from jax.experimental import pallas as pl
from jax.experimental.pallas import tpu as pltpu
from jax.experimental.pallas import tpu_sc as plsc

N = 10000
NS = 10000
E = 160000
ECG = 160000
D = 256
DE = 16
DEO = 128
DU = 64
DA = 64
B = 4

BM = 2000
BE = 2000
DAP = 128


def _proj_body(x_ref, xs_ref, wcat_ref, wk_ref, xcat_ref, k_ref):
    xcat_ref[...] = jnp.dot(x_ref[...], wcat_ref[...],
                            preferred_element_type=jnp.float32)
    k_ref[...] = jnp.dot(xs_ref[...], wk_ref[...],
                         preferred_element_type=jnp.float32)


def _proj(x, x_src, Wqp, Wkp):
    return pl.pallas_call(
        _proj_body,
        grid=(N // BM,),
        in_specs=[
            pl.BlockSpec((BM, D), lambda i: (i, 0)),
            pl.BlockSpec((BM, D), lambda i: (i, 0)),
            pl.BlockSpec((D, DAP), lambda i: (0, 0)),
            pl.BlockSpec((D, DAP), lambda i: (0, 0)),
        ],
        out_specs=[
            pl.BlockSpec((BM, DAP), lambda i: (i, 0)),
            pl.BlockSpec((BM, DAP), lambda i: (i, 0)),
        ],
        out_shape=[
            jax.ShapeDtypeStruct((N, DAP), jnp.float32),
            jax.ShapeDtypeStruct((NS, DAP), jnp.float32),
        ],
    )(x, x_src, Wqp, Wkp)


def _projx_body(x_ref, w_ref, o_ref):
    o_ref[...] = jnp.dot(x_ref[...], w_ref[...],
                         preferred_element_type=jnp.float32)


def _proj_x(x, Wcat):
    DC = Wcat.shape[1]
    return pl.pallas_call(
        _projx_body,
        grid=(N // BM,),
        in_specs=[
            pl.BlockSpec((BM, D), lambda i: (i, 0)),
            pl.BlockSpec((D, DC), lambda i: (0, 0)),
        ],
        out_specs=pl.BlockSpec((BM, DC), lambda i: (i, 0)),
        out_shape=jax.ShapeDtypeStruct((N, DC), jnp.float32),
    )(x, Wcat)


def _tables_body(a_ref, xr_ref, xc_ref, xn_ref, un_ref, un2_ref,
                 pr_ref, pc_ref, pn_ref):
    av = a_ref[...]
    pr_ref[...] = av * xr_ref[...] + un_ref[...]
    pc_ref[...] = av * xc_ref[...]
    pn_ref[...] = av * xn_ref[...] + un2_ref[...]


def _tables(A, Xr, Xc, Xn, Un, Un2):
    spec = pl.BlockSpec((BM, D), lambda i: (i, 0))
    return pl.pallas_call(
        _tables_body,
        grid=(N // BM,),
        in_specs=[spec] * 6,
        out_specs=[spec] * 3,
        out_shape=[jax.ShapeDtypeStruct((N, D), jnp.float32)] * 3,
    )(A, Xr, Xc, Xn, Un, Un2)


def _edge_body(g_ref, ea_ref, we1e_ref, we2_ref, be2_ref, out_ref):
    h = g_ref[...] + jnp.dot(ea_ref[...], we1e_ref[...],
                             preferred_element_type=jnp.float32)
    h = jnp.maximum(h, 0.0)
    out_ref[...] = jnp.dot(h, we2_ref[...],
                           preferred_element_type=jnp.float32) + be2_ref[...]


def _edge_mlp(G, edge_attr, We1e, We2, be2row):
    return pl.pallas_call(
        _edge_body,
        grid=(E // BE,),
        in_specs=[
            pl.BlockSpec((BE, D), lambda i: (i, 0)),
            pl.BlockSpec((BE, DE), lambda i: (i, 0)),
            pl.BlockSpec((DE, D), lambda i: (0, 0)),
            pl.BlockSpec((D, DEO), lambda i: (0, 0)),
            pl.BlockSpec((1, DEO), lambda i: (0, 0)),
        ],
        out_specs=pl.BlockSpec((BE, DEO), lambda i: (i, 0)),
        out_shape=jax.ShapeDtypeStruct((E, DEO), jnp.float32),
    )(G, edge_attr, We1e, We2, be2row)


def _node_body(pn_ref, aggm_ref, wn1a_ref, wn2_ref, bn2_ref, bh_ref,
               u_ref, wg1p_ref, wg1u_ref, bg1_ref, wg2_ref, bg2_ref,
               pcinv_ref, xout_ref, uout_ref, psum_scr):
    i = pl.program_id(0)
    npre = pn_ref[...] + jnp.dot(aggm_ref[...], wn1a_ref[...],
                                 preferred_element_type=jnp.float32)
    h = jnp.maximum(npre, 0.0)
    xout = jnp.dot(h, wn2_ref[...],
                   preferred_element_type=jnp.float32) + bn2_ref[...]
    xout_ref[...] = xout

    part = lax.dot_general(bh_ref[...], xout, (((0,), (0,)), ((), ())),
                           preferred_element_type=jnp.float32)

    @pl.when(i == 0)
    def _init():
        psum_scr[...] = jnp.zeros_like(psum_scr)

    psum_scr[:B, :] += part

    @pl.when(i == pl.num_programs(0) - 1)
    def _fin():
        pool = psum_scr[:B, :] * pcinv_ref[...]
        h2 = jnp.dot(pool, wg1p_ref[...], preferred_element_type=jnp.float32)
        h2 = h2 + jnp.dot(u_ref[...], wg1u_ref[...],
                          preferred_element_type=jnp.float32) + bg1_ref[...]
        h2 = jnp.maximum(h2, 0.0)
        uout_ref[...] = jnp.dot(h2, wg2_ref[...],
                                preferred_element_type=jnp.float32) + bg2_ref[...]


def _node_global(Pn, AggM, Wn1a, Wn2, bn2row, Bh, u, Wg1p, Wg1u, bg1row,
                 Wg2, bg2row, PcInv):
    return pl.pallas_call(
        _node_body,
        grid=(N // BM,),
        in_specs=[
            pl.BlockSpec((BM, D), lambda i: (i, 0)),
            pl.BlockSpec((BM, DEO), lambda i: (i, 0)),
            pl.BlockSpec((DEO, D), lambda i: (0, 0)),
            pl.BlockSpec((D, D), lambda i: (0, 0)),
            pl.BlockSpec((1, D), lambda i: (0, 0)),
            pl.BlockSpec((BM, B), lambda i: (i, 0)),
            pl.BlockSpec((B, DU), lambda i: (0, 0)),
            pl.BlockSpec((D, 128), lambda i: (0, 0)),
            pl.BlockSpec((DU, 128), lambda i: (0, 0)),
            pl.BlockSpec((1, 128), lambda i: (0, 0)),
            pl.BlockSpec((128, DU), lambda i: (0, 0)),
            pl.BlockSpec((1, DU), lambda i: (0, 0)),
            pl.BlockSpec((B, D), lambda i: (0, 0)),
        ],
        out_specs=[
            pl.BlockSpec((BM, D), lambda i: (i, 0)),
            pl.BlockSpec((B, DU), lambda i: (0, 0)),
        ],
        out_shape=[
            jax.ShapeDtypeStruct((N, D), jnp.float32),
            jax.ShapeDtypeStruct((B, DU), jnp.float32),
        ],
        scratch_shapes=[pltpu.VMEM((8, D), jnp.float32)],
    )(Pn, AggM, Wn1a, Wn2, bn2row, Bh, u, Wg1p, Wg1u, bg1row, Wg2, bg2row,
      PcInv)


NC = 2
NSUB = 16
NW = NC * NSUB
CG = 128
NCHUNK = E // CG
EPW = E // NW


def _sc_mesh():
    return plsc.VectorSubcoreMesh(core_axis_name="c", subcore_axis_name="s",
                                  num_cores=NC, num_subcores=NSUB)


def _worker_id():
    return lax.axis_index("s") * NC + lax.axis_index("c")


NPAD = 10240
TROWS = NPAD // NSUB


NCH2 = (EPW + CG - 1) // CG
DUP2 = NCH2 * CG - EPW


def _attn_body(q_hbm, k_hbm, ci_hbm, cj_hbm, ssum_hbm, cnt_hbm,
               ci_all, cj_all, ci_m0, ci_m1, qbuf0, kbuf0, qbuf1, kbuf1,
               sbuf, obuf, cbuf, sh_s, sh_c, sem0, sem1):
    core = lax.axis_index("c")
    sid = lax.axis_index("s")
    w = sid * NC + core
    base = w * EPW
    pltpu.sync_copy(ci_hbm.at[pl.ds(base, EPW)], ci_all)
    pltpu.sync_copy(cj_hbm.at[pl.ds(base, EPW)], cj_all)

    def fill(r, _):
        sl = pl.ds(r * 16, 16)
        obuf[sl] = jnp.full((16,), 1.0, jnp.float32)
        cbuf[sl] = jnp.zeros((16,), jnp.float32)
        return 0

    lax.fori_loop(0, CG // 16, fill, 0)

    def zinit(t, _):
        sl = pl.ds(sid * TROWS + t * CG, CG)
        pltpu.sync_copy(cbuf, sh_s.at[sl])
        pltpu.sync_copy(cbuf, sh_c.at[sl])
        return 0

    lax.fori_loop(0, TROWS // CG, zinit, 0)
    plsc.subcore_barrier()

    lanes = jnp.arange(16, dtype=jnp.int32)
    bufs = ((qbuf0, kbuf0, ci_m0, sem0), (qbuf1, kbuf1, ci_m1, sem1))

    def woff(c):
        return jnp.minimum(c * CG, EPW - CG)

    def issue(c, bset):
        qbuf, kbuf, ci_m, sem = bset
        o = woff(c)
        def mkidx(g, _):
            sl = pl.ds(o + g * 16, 16)
            idxv = ci_all[sl]
            dup = (c == NCH2 - 1) & (g * 16 + lanes < DUP2)
            ci_m[pl.ds(g * 16, 16)] = jnp.where(
                dup, jnp.full((16,), NPAD - 1, jnp.int32), idxv)
            return 0

        lax.fori_loop(0, CG // 16, mkidx, 0)
        pltpu.async_copy(q_hbm.at[ci_all.at[pl.ds(o, CG)]], qbuf, sem)
        pltpu.async_copy(k_hbm.at[cj_all.at[pl.ds(o, CG)]], kbuf, sem)

    def finish(c, bset):
        qbuf, kbuf, ci_m, sem = bset
        o = woff(c)
        pltpu.make_async_copy(
            q_hbm.at[ci_all.at[pl.ds(o, CG)]], qbuf, sem).wait()
        pltpu.make_async_copy(
            k_hbm.at[cj_all.at[pl.ds(o, CG)]], kbuf, sem).wait()

        def group(g, _):
            v = jnp.zeros((16,), jnp.float32)
            for j in range(16):
                e = g * 16 + j
                acc = jnp.zeros((16,), jnp.float32)
                for cdim in range(DA // 16):
                    sl = pl.ds(cdim * 16, 16)
                    acc = acc + qbuf[e, sl] * kbuf[e, sl]
                s = jnp.sum(acc)
                v = jnp.where(lanes == j, jnp.full((16,), s), v)
            v = 1.0 / (1.0 + jnp.exp(v * -0.125))
            sbuf[pl.ds(g * 16, 16)] = v
            return 0

        lax.fori_loop(0, CG // 16, group, 0)
        pltpu.sync_copy(sbuf, sh_s.at[ci_m], add=True)
        pltpu.sync_copy(obuf, sh_c.at[ci_m], add=True)

    issue(0, bufs[0])

    def step(t2, _):
        for b in range(2):
            c = t2 * 2 + b

            @pl.when(c < NCH2)
            def _do():
                @pl.when(c + 1 < NCH2)
                def _nxt():
                    issue(c + 1, bufs[1 - b])

                finish(c, bufs[b])

        return 0

    lax.fori_loop(0, (NCH2 + 1) // 2, step, 0)
    plsc.subcore_barrier()

    def out(t, _):
        sl = pl.ds(sid * TROWS + t * CG, CG)
        pltpu.sync_copy(sh_s.at[sl], cbuf)
        pltpu.sync_copy(cbuf, ssum_hbm.at[core, sl])
        pltpu.sync_copy(sh_c.at[sl], cbuf)
        pltpu.sync_copy(cbuf, cnt_hbm.at[core, sl])
        return 0

    lax.fori_loop(0, TROWS // CG, out, 0)


def _attention_sums(q, k, ci, cj):
    call = pl.kernel(
        _attn_body,
        out_type=(jax.ShapeDtypeStruct((NC, NPAD), jnp.float32),
                  jax.ShapeDtypeStruct((NC, NPAD), jnp.float32)),
        mesh=_sc_mesh(),
        compiler_params=pltpu.CompilerParams(needs_layout_passes=False),
        scratch_types=[
            pltpu.VMEM((EPW,), jnp.int32),
            pltpu.VMEM((EPW,), jnp.int32),
            pltpu.VMEM((CG,), jnp.int32),
            pltpu.VMEM((CG,), jnp.int32),
            pltpu.VMEM((CG, DAP), jnp.float32),
            pltpu.VMEM((CG, DAP), jnp.float32),
            pltpu.VMEM((CG, DAP), jnp.float32),
            pltpu.VMEM((CG, DAP), jnp.float32),
            pltpu.VMEM((CG,), jnp.float32),
            pltpu.VMEM((CG,), jnp.float32),
            pltpu.VMEM((CG,), jnp.float32),
            pltpu.VMEM_SHARED((NPAD,), jnp.float32),
            pltpu.VMEM_SHARED((NPAD,), jnp.float32),
            pltpu.SemaphoreType.DMA,
            pltpu.SemaphoreType.DMA,
        ],
    )
    S, C = call(q, k, ci, cj)
    ssum = S[0, :N] + S[1, :N]
    cnt = C[0, :N] + C[1, :N]
    return ssum, cnt


CG4 = 80
NCH4 = (EPW + CG4 - 1) // CG4


def _gather_body(pr_hbm, pc_hbm, row_hbm, col_hbm, g_hbm,
                 ridx_all, cidx_all, rbuf0, cbuf0, rbuf1, cbuf1,
                 sem0, sem1):
    w = _worker_id()
    base = w * EPW
    pltpu.sync_copy(row_hbm.at[pl.ds(base, EPW)], ridx_all)
    pltpu.sync_copy(col_hbm.at[pl.ds(base, EPW)], cidx_all)

    bufs = ((rbuf0, cbuf0, sem0), (rbuf1, cbuf1, sem1))

    def woff(c):
        return jnp.minimum(c * CG4, EPW - CG4)

    def issue(c, bset):
        rbuf, cbuf, sem = bset
        o = woff(c)
        pltpu.async_copy(pr_hbm.at[ridx_all.at[pl.ds(o, CG4)]], rbuf, sem)
        pltpu.async_copy(pc_hbm.at[cidx_all.at[pl.ds(o, CG4)]], cbuf, sem)

    def finish(c, bset):
        rbuf, cbuf, sem = bset
        o = woff(c)
        pltpu.make_async_copy(
            pr_hbm.at[ridx_all.at[pl.ds(o, CG4)]], rbuf, sem).wait()
        pltpu.make_async_copy(
            pc_hbm.at[cidx_all.at[pl.ds(o, CG4)]], cbuf, sem).wait()

        def add_row(r, _):
            for j in range(D // 16):
                sl = pl.ds(j * 16, 16)
                rbuf[r, sl] = rbuf[r, sl] + cbuf[r, sl]
            return 0

        lax.fori_loop(0, CG4, add_row, 0)
        pltpu.sync_copy(rbuf, g_hbm.at[pl.ds(base + o, CG4)])

    issue(0, bufs[0])

    def step(t2, _):
        for b in range(2):
            c = t2 * 2 + b

            @pl.when(c < NCH4)
            def _do():
                @pl.when(c + 1 < NCH4)
                def _nxt():
                    issue(c + 1, bufs[1 - b])

                finish(c, bufs[b])

        return 0

    lax.fori_loop(0, (NCH4 + 1) // 2, step, 0)


def _gather_sum(Pr, Pc, row, col):
    call = pl.kernel(
        _gather_body,
        out_type=jax.ShapeDtypeStruct((E, D), jnp.float32),
        mesh=_sc_mesh(),
        compiler_params=pltpu.CompilerParams(needs_layout_passes=False),
        scratch_types=[
            pltpu.VMEM((EPW,), jnp.int32),
            pltpu.VMEM((EPW,), jnp.int32),
            pltpu.VMEM((CG4, D), jnp.float32),
            pltpu.VMEM((CG4, D), jnp.float32),
            pltpu.VMEM((CG4, D), jnp.float32),
            pltpu.VMEM((CG4, D), jnp.float32),
            pltpu.SemaphoreType.DMA,
            pltpu.SemaphoreType.DMA,
        ],
    )
    return call(Pr, Pc, row, col)


CG6 = 40
NCH6 = EPW // CG6


def _scatter_body(eout_hbm, row_hbm, agg_hbm, ecnt_hbm,
                  ridx0, ridx1, ebuf0, ebuf1, obuf, zbuf, cbuf,
                  sh_agg, sh_cnt, sem0, sem1):
    core = lax.axis_index("c")
    sid = lax.axis_index("s")
    w = sid * NC + core
    base = w * EPW

    def zrow(r, _):
        for j in range(DEO // 16):
            zbuf[r, pl.ds(j * 16, 16)] = jnp.zeros((16,), jnp.float32)
        return 0

    lax.fori_loop(0, 128, zrow, 0)

    def fillc(r, _):
        cbuf[pl.ds(r * 16, 16)] = jnp.zeros((16,), jnp.float32)
        return 0

    lax.fori_loop(0, 8, fillc, 0)

    def zinit(t, _):
        sl = pl.ds(sid * TROWS + t * 128, 128)
        pltpu.sync_copy(zbuf, sh_agg.at[sl])
        pltpu.sync_copy(cbuf, sh_cnt.at[sl])
        return 0

    lax.fori_loop(0, TROWS // 128, zinit, 0)
    for o in (0, 16, CG6 - 16):
        obuf[pl.ds(o, 16)] = jnp.full((16,), 1.0, jnp.float32)
    plsc.subcore_barrier()

    bufs = ((ebuf0, ridx0, sem0), (ebuf1, ridx1, sem1))

    def issue(c, bset):
        ebuf, ridx, sem = bset
        o = base + c * CG6
        pltpu.async_copy(row_hbm.at[pl.ds(o, CG6)], ridx, sem)
        pltpu.async_copy(eout_hbm.at[pl.ds(o, CG6)], ebuf, sem)

    def finish(c, bset):
        ebuf, ridx, sem = bset
        o = base + c * CG6
        pltpu.make_async_copy(row_hbm.at[pl.ds(o, CG6)], ridx, sem).wait()
        pltpu.make_async_copy(eout_hbm.at[pl.ds(o, CG6)], ebuf, sem).wait()
        pltpu.sync_copy(ebuf, sh_agg.at[ridx], add=True)
        pltpu.sync_copy(obuf, sh_cnt.at[ridx], add=True)

    issue(0, bufs[0])

    def step(t2, _):
        for b in range(2):
            c = t2 * 2 + b

            @pl.when(c < NCH6)
            def _do():
                @pl.when(c + 1 < NCH6)
                def _nxt():
                    issue(c + 1, bufs[1 - b])

                finish(c, bufs[b])

        return 0

    lax.fori_loop(0, (NCH6 + 1) // 2, step, 0)
    plsc.subcore_barrier()

    def out(t, _):
        sl = pl.ds(sid * TROWS + t * 128, 128)
        pltpu.sync_copy(sh_agg.at[sl], zbuf)
        pltpu.sync_copy(zbuf, agg_hbm.at[core, sl])
        pltpu.sync_copy(sh_cnt.at[sl], cbuf)
        pltpu.sync_copy(cbuf, ecnt_hbm.at[core, sl])
        return 0

    lax.fori_loop(0, TROWS // 128, out, 0)


def _scatter_mean(eout, row):
    call = pl.kernel(
        _scatter_body,
        out_type=(jax.ShapeDtypeStruct((NC, NPAD, DEO), jnp.float32),
                  jax.ShapeDtypeStruct((NC, NPAD), jnp.float32)),
        mesh=_sc_mesh(),
        compiler_params=pltpu.CompilerParams(needs_layout_passes=False),
        scratch_types=[
            pltpu.VMEM((CG6,), jnp.int32),
            pltpu.VMEM((CG6,), jnp.int32),
            pltpu.VMEM((CG6, DEO), jnp.float32),
            pltpu.VMEM((CG6, DEO), jnp.float32),
            pltpu.VMEM((CG6,), jnp.float32),
            pltpu.VMEM((128, DEO), jnp.float32),
            pltpu.VMEM((128,), jnp.float32),
            pltpu.VMEM_SHARED((NPAD, DEO), jnp.float32),
            pltpu.VMEM_SHARED((NPAD,), jnp.float32),
            pltpu.SemaphoreType.DMA,
            pltpu.SemaphoreType.DMA,
        ],
    )
    A, C = call(eout, row)
    agg = A[0, :N] + A[1, :N]
    ecnt = C[0, :N] + C[1, :N]
    return agg / jnp.maximum(ecnt, 1.0)[:, None]


def kernel(x, x_src, edge_index, cg_edge_index, edge_attr, u, batch,
           batch_src, Wq, Wk, We1, be1, We2, be2, Wn1, bn1, Wn2, bn2,
           Wg1, bg1, Wg2, bg2):
    We1_r = We1[:D]
    We1_c = We1[D:2 * D]
    We1_e = We1[2 * D:2 * D + DE]
    We1_u = We1[2 * D + DE:]
    Wn1_x = Wn1[:D]
    Wn1_a = Wn1[D:D + DEO]
    Wn1_u = Wn1[D + DEO:]

    Wqp = jnp.pad(Wq, ((0, 0), (0, DAP - DA)))
    Wkp = jnp.pad(Wk, ((0, 0), (0, DAP - DA)))
    q, k = _proj(x, x_src, Wqp, Wkp)

    ci = cg_edge_index[0]
    cj = cg_edge_index[1]
    ssum, cnt = _attention_sums(q, k, ci, cj)
    a = ssum / jnp.maximum(cnt, 1.0)

    Wcat = jnp.concatenate([We1_r, We1_c, Wn1_x], axis=1)
    Xcat = _proj_x(x, Wcat)
    Xr = Xcat[:, :D]
    Xc = Xcat[:, D:2 * D]
    Xn = Xcat[:, 2 * D:]

    A = jnp.broadcast_to(a[:, None], (N, D))
    Un = (u @ We1_u + be1)[batch]
    Un2 = (u @ Wn1_u + bn1)[batch]
    Pr, Pc, Pn = _tables(A, Xr, Xc, Xn, Un, Un2)

    row = edge_index[0]
    col = edge_index[1]
    G = _gather_sum(Pr, Pc, row, col)

    eout = _edge_mlp(G, edge_attr, We1_e, We2, be2.reshape(1, DEO))

    AggM = _scatter_mean(eout, row)

    Bh = jax.nn.one_hot(batch, B, dtype=jnp.float32)
    pcnt = jnp.maximum(jnp.sum(Bh, axis=0), 1.0)
    PcInv = jnp.broadcast_to((1.0 / pcnt)[:, None], (B, D))
    x_out, u_out = _node_global(Pn, AggM, Wn1_a, Wn2, bn2.reshape(1, D),
                                Bh, u, Wg1[:D], Wg1[D:], bg1.reshape(1, 128),
                                Wg2, bg2.reshape(1, DU), PcInv)
    return (x_out, eout, u_out)


# speedup vs baseline: 6.3175x; 1.0442x over previous
import jax
import jax.numpy as jnp
from jax import lax
from jax.experimental import pallas as pl
from jax.experimental.pallas import tpu as pltpu
from jax.experimental.pallas import tpu_sc as plsc

N = 10000
NS = 10000
E = 160000
ECG = 160000
D = 256
DE = 16
DEO = 128
DU = 64
DA = 64
B = 4

BM = 2000
BE = 2000
DAP = 128


def _proj_body(x_ref, xs_ref, wcat_ref, wk_ref, xcat_ref, k_ref):
    xcat_ref[...] = jnp.dot(x_ref[...], wcat_ref[...],
                            preferred_element_type=jnp.float32)
    k_ref[...] = jnp.dot(xs_ref[...], wk_ref[...],
                         preferred_element_type=jnp.float32)


def _proj(x, x_src, Wqp, Wkp):
    return pl.pallas_call(
        _proj_body,
        grid=(N // BM,),
        in_specs=[
            pl.BlockSpec((BM, D), lambda i: (i, 0)),
            pl.BlockSpec((BM, D), lambda i: (i, 0)),
            pl.BlockSpec((D, DAP), lambda i: (0, 0)),
            pl.BlockSpec((D, DAP), lambda i: (0, 0)),
        ],
        out_specs=[
            pl.BlockSpec((BM, DAP), lambda i: (i, 0)),
            pl.BlockSpec((BM, DAP), lambda i: (i, 0)),
        ],
        out_shape=[
            jax.ShapeDtypeStruct((N, DAP), jnp.float32),
            jax.ShapeDtypeStruct((NS, DAP), jnp.float32),
        ],
    )(x, x_src, Wqp, Wkp)


def _tables_body(x_ref, w_ref, a_ref, un_ref, un2_ref,
                 pr_ref, pc_ref, pn_ref):
    xcat = jnp.dot(x_ref[...], w_ref[...],
                   preferred_element_type=jnp.float32)
    av = a_ref[...]
    pr_ref[...] = av * xcat[:, :D] + un_ref[...]
    pc_ref[...] = av * xcat[:, D:2 * D]
    pn_ref[...] = av * xcat[:, 2 * D:] + un2_ref[...]


def _tables(x, Wcat, A, Un, Un2):
    spec = pl.BlockSpec((BM, D), lambda i: (i, 0))
    return pl.pallas_call(
        _tables_body,
        grid=(N // BM,),
        in_specs=[
            spec,
            pl.BlockSpec((D, 3 * D), lambda i: (0, 0)),
            spec,
            spec,
            spec,
        ],
        out_specs=[spec] * 3,
        out_shape=[jax.ShapeDtypeStruct((N, D), jnp.float32)] * 3,
    )(x, Wcat, A, Un, Un2)


def _edge_body(g_ref, ea_ref, we1e_ref, we2_ref, be2_ref, out_ref):
    h = g_ref[...] + jnp.dot(ea_ref[...], we1e_ref[...],
                             preferred_element_type=jnp.float32)
    h = jnp.maximum(h, 0.0)
    out_ref[...] = jnp.dot(h, we2_ref[...],
                           preferred_element_type=jnp.float32) + be2_ref[...]


def _edge_mlp(G, edge_attr, We1e, We2, be2row):
    return pl.pallas_call(
        _edge_body,
        grid=(E // BE,),
        in_specs=[
            pl.BlockSpec((BE, D), lambda i: (i, 0)),
            pl.BlockSpec((BE, DE), lambda i: (i, 0)),
            pl.BlockSpec((DE, D), lambda i: (0, 0)),
            pl.BlockSpec((D, DEO), lambda i: (0, 0)),
            pl.BlockSpec((1, DEO), lambda i: (0, 0)),
        ],
        out_specs=pl.BlockSpec((BE, DEO), lambda i: (i, 0)),
        out_shape=jax.ShapeDtypeStruct((E, DEO), jnp.float32),
    )(G, edge_attr, We1e, We2, be2row)


def _node_body(pn_ref, aggm_ref, wn1a_ref, wn2_ref, bn2_ref, bh_ref,
               u_ref, wg1p_ref, wg1u_ref, bg1_ref, wg2_ref, bg2_ref,
               pcinv_ref, xout_ref, uout_ref, psum_scr):
    i = pl.program_id(0)
    npre = pn_ref[...] + jnp.dot(aggm_ref[...], wn1a_ref[...],
                                 preferred_element_type=jnp.float32)
    h = jnp.maximum(npre, 0.0)
    xout = jnp.dot(h, wn2_ref[...],
                   preferred_element_type=jnp.float32) + bn2_ref[...]
    xout_ref[...] = xout

    part = lax.dot_general(bh_ref[...], xout, (((0,), (0,)), ((), ())),
                           preferred_element_type=jnp.float32)

    @pl.when(i == 0)
    def _init():
        psum_scr[...] = jnp.zeros_like(psum_scr)

    psum_scr[:B, :] += part

    @pl.when(i == pl.num_programs(0) - 1)
    def _fin():
        pool = psum_scr[:B, :] * pcinv_ref[...]
        h2 = jnp.dot(pool, wg1p_ref[...], preferred_element_type=jnp.float32)
        h2 = h2 + jnp.dot(u_ref[...], wg1u_ref[...],
                          preferred_element_type=jnp.float32) + bg1_ref[...]
        h2 = jnp.maximum(h2, 0.0)
        uout_ref[...] = jnp.dot(h2, wg2_ref[...],
                                preferred_element_type=jnp.float32) + bg2_ref[...]


def _node_global(Pn, AggM, Wn1a, Wn2, bn2row, Bh, u, Wg1p, Wg1u, bg1row,
                 Wg2, bg2row, PcInv):
    return pl.pallas_call(
        _node_body,
        grid=(N // BM,),
        in_specs=[
            pl.BlockSpec((BM, D), lambda i: (i, 0)),
            pl.BlockSpec((BM, DEO), lambda i: (i, 0)),
            pl.BlockSpec((DEO, D), lambda i: (0, 0)),
            pl.BlockSpec((D, D), lambda i: (0, 0)),
            pl.BlockSpec((1, D), lambda i: (0, 0)),
            pl.BlockSpec((BM, B), lambda i: (i, 0)),
            pl.BlockSpec((B, DU), lambda i: (0, 0)),
            pl.BlockSpec((D, 128), lambda i: (0, 0)),
            pl.BlockSpec((DU, 128), lambda i: (0, 0)),
            pl.BlockSpec((1, 128), lambda i: (0, 0)),
            pl.BlockSpec((128, DU), lambda i: (0, 0)),
            pl.BlockSpec((1, DU), lambda i: (0, 0)),
            pl.BlockSpec((B, D), lambda i: (0, 0)),
        ],
        out_specs=[
            pl.BlockSpec((BM, D), lambda i: (i, 0)),
            pl.BlockSpec((B, DU), lambda i: (0, 0)),
        ],
        out_shape=[
            jax.ShapeDtypeStruct((N, D), jnp.float32),
            jax.ShapeDtypeStruct((B, DU), jnp.float32),
        ],
        scratch_shapes=[pltpu.VMEM((8, D), jnp.float32)],
    )(Pn, AggM, Wn1a, Wn2, bn2row, Bh, u, Wg1p, Wg1u, bg1row, Wg2, bg2row,
      PcInv)


NC = 2
NSUB = 16
NW = NC * NSUB
CG = 128
NCHUNK = E // CG
EPW = E // NW


def _sc_mesh():
    return plsc.VectorSubcoreMesh(core_axis_name="c", subcore_axis_name="s",
                                  num_cores=NC, num_subcores=NSUB)


def _worker_id():
    return lax.axis_index("s") * NC + lax.axis_index("c")


NPAD = 10240
TROWS = NPAD // NSUB


NCH2 = (EPW + CG - 1) // CG
DUP2 = NCH2 * CG - EPW


def _attn_body(q_hbm, k_hbm, ci_hbm, cj_hbm, ssum_hbm, cnt_hbm,
               ci_all, cj_all, ci_m0, ci_m1, qbuf0, kbuf0, qbuf1, kbuf1,
               sbuf, obuf, cbuf, sh_s, sh_c, sem0, sem1):
    core = lax.axis_index("c")
    sid = lax.axis_index("s")
    w = sid * NC + core
    base = w * EPW
    pltpu.sync_copy(ci_hbm.at[pl.ds(base, EPW)], ci_all)
    pltpu.sync_copy(cj_hbm.at[pl.ds(base, EPW)], cj_all)

    def fill(r, _):
        sl = pl.ds(r * 16, 16)
        obuf[sl] = jnp.full((16,), 1.0, jnp.float32)
        cbuf[sl] = jnp.zeros((16,), jnp.float32)
        return 0

    lax.fori_loop(0, CG // 16, fill, 0)

    def zinit(t, _):
        sl = pl.ds(sid * TROWS + t * CG, CG)
        pltpu.sync_copy(cbuf, sh_s.at[sl])
        pltpu.sync_copy(cbuf, sh_c.at[sl])
        return 0

    lax.fori_loop(0, TROWS // CG, zinit, 0)
    plsc.subcore_barrier()

    lanes = jnp.arange(16, dtype=jnp.int32)
    bufs = ((qbuf0, kbuf0, ci_m0, sem0), (qbuf1, kbuf1, ci_m1, sem1))

    def woff(c):
        return jnp.minimum(c * CG, EPW - CG)

    def issue(c, bset):
        qbuf, kbuf, ci_m, sem = bset
        o = woff(c)
        def mkidx(g, _):
            sl = pl.ds(o + g * 16, 16)
            idxv = ci_all[sl]
            dup = (c == NCH2 - 1) & (g * 16 + lanes < DUP2)
            ci_m[pl.ds(g * 16, 16)] = jnp.where(
                dup, jnp.full((16,), NPAD - 1, jnp.int32), idxv)
            return 0

        lax.fori_loop(0, CG // 16, mkidx, 0)
        pltpu.async_copy(q_hbm.at[ci_all.at[pl.ds(o, CG)]], qbuf, sem)
        pltpu.async_copy(k_hbm.at[cj_all.at[pl.ds(o, CG)]], kbuf, sem)

    def finish(c, bset):
        qbuf, kbuf, ci_m, sem = bset
        o = woff(c)
        pltpu.make_async_copy(
            q_hbm.at[ci_all.at[pl.ds(o, CG)]], qbuf, sem).wait()
        pltpu.make_async_copy(
            k_hbm.at[cj_all.at[pl.ds(o, CG)]], kbuf, sem).wait()

        def group(g, _):
            v = jnp.zeros((16,), jnp.float32)
            for j in range(16):
                e = g * 16 + j
                acc = jnp.zeros((16,), jnp.float32)
                for cdim in range(DA // 16):
                    sl = pl.ds(cdim * 16, 16)
                    acc = acc + qbuf[e, sl] * kbuf[e, sl]
                s = jnp.sum(acc)
                v = jnp.where(lanes == j, jnp.full((16,), s), v)
            v = 1.0 / (1.0 + jnp.exp(v * -0.125))
            sbuf[pl.ds(g * 16, 16)] = v
            return 0

        lax.fori_loop(0, CG // 16, group, 0)
        pltpu.sync_copy(sbuf, sh_s.at[ci_m], add=True)
        pltpu.sync_copy(obuf, sh_c.at[ci_m], add=True)

    issue(0, bufs[0])

    def step(t2, _):
        for b in range(2):
            c = t2 * 2 + b

            @pl.when(c < NCH2)
            def _do():
                @pl.when(c + 1 < NCH2)
                def _nxt():
                    issue(c + 1, bufs[1 - b])

                finish(c, bufs[b])

        return 0

    lax.fori_loop(0, (NCH2 + 1) // 2, step, 0)
    plsc.subcore_barrier()

    def out(t, _):
        sl = pl.ds(sid * TROWS + t * CG, CG)
        pltpu.sync_copy(sh_s.at[sl], cbuf)
        pltpu.sync_copy(cbuf, ssum_hbm.at[core, sl])
        pltpu.sync_copy(sh_c.at[sl], cbuf)
        pltpu.sync_copy(cbuf, cnt_hbm.at[core, sl])
        return 0

    lax.fori_loop(0, TROWS // CG, out, 0)


def _attention_sums(q, k, ci, cj):
    call = pl.kernel(
        _attn_body,
        out_type=(jax.ShapeDtypeStruct((NC, NPAD), jnp.float32),
                  jax.ShapeDtypeStruct((NC, NPAD), jnp.float32)),
        mesh=_sc_mesh(),
        compiler_params=pltpu.CompilerParams(needs_layout_passes=False),
        scratch_types=[
            pltpu.VMEM((EPW,), jnp.int32),
            pltpu.VMEM((EPW,), jnp.int32),
            pltpu.VMEM((CG,), jnp.int32),
            pltpu.VMEM((CG,), jnp.int32),
            pltpu.VMEM((CG, DAP), jnp.float32),
            pltpu.VMEM((CG, DAP), jnp.float32),
            pltpu.VMEM((CG, DAP), jnp.float32),
            pltpu.VMEM((CG, DAP), jnp.float32),
            pltpu.VMEM((CG,), jnp.float32),
            pltpu.VMEM((CG,), jnp.float32),
            pltpu.VMEM((CG,), jnp.float32),
            pltpu.VMEM_SHARED((NPAD,), jnp.float32),
            pltpu.VMEM_SHARED((NPAD,), jnp.float32),
            pltpu.SemaphoreType.DMA,
            pltpu.SemaphoreType.DMA,
        ],
    )
    S, C = call(q, k, ci, cj)
    ssum = S[0, :N] + S[1, :N]
    cnt = C[0, :N] + C[1, :N]
    return ssum, cnt


CG4 = 80
NCH4 = (EPW + CG4 - 1) // CG4


def _gather_body(pr_hbm, pc_hbm, row_hbm, col_hbm, g_hbm,
                 ridx_all, cidx_all, rbuf0, cbuf0, rbuf1, cbuf1,
                 sem0, sem1):
    w = _worker_id()
    base = w * EPW
    pltpu.sync_copy(row_hbm.at[pl.ds(base, EPW)], ridx_all)
    pltpu.sync_copy(col_hbm.at[pl.ds(base, EPW)], cidx_all)

    bufs = ((rbuf0, cbuf0, sem0), (rbuf1, cbuf1, sem1))

    def woff(c):
        return jnp.minimum(c * CG4, EPW - CG4)

    def issue(c, bset):
        rbuf, cbuf, sem = bset
        o = woff(c)
        pltpu.async_copy(pr_hbm.at[ridx_all.at[pl.ds(o, CG4)]], rbuf, sem)
        pltpu.async_copy(pc_hbm.at[cidx_all.at[pl.ds(o, CG4)]], cbuf, sem)

    def finish(c, bset):
        rbuf, cbuf, sem = bset
        o = woff(c)
        pltpu.make_async_copy(
            pr_hbm.at[ridx_all.at[pl.ds(o, CG4)]], rbuf, sem).wait()
        pltpu.make_async_copy(
            pc_hbm.at[cidx_all.at[pl.ds(o, CG4)]], cbuf, sem).wait()

        def add_row(r, _):
            for j in range(D // 16):
                sl = pl.ds(j * 16, 16)
                rbuf[r, sl] = rbuf[r, sl] + cbuf[r, sl]
            return 0

        lax.fori_loop(0, CG4, add_row, 0)
        pltpu.sync_copy(rbuf, g_hbm.at[pl.ds(base + o, CG4)])

    issue(0, bufs[0])

    def step(t2, _):
        for b in range(2):
            c = t2 * 2 + b

            @pl.when(c < NCH4)
            def _do():
                @pl.when(c + 1 < NCH4)
                def _nxt():
                    issue(c + 1, bufs[1 - b])

                finish(c, bufs[b])

        return 0

    lax.fori_loop(0, (NCH4 + 1) // 2, step, 0)


def _gather_sum(Pr, Pc, row, col):
    call = pl.kernel(
        _gather_body,
        out_type=jax.ShapeDtypeStruct((E, D), jnp.float32),
        mesh=_sc_mesh(),
        compiler_params=pltpu.CompilerParams(needs_layout_passes=False),
        scratch_types=[
            pltpu.VMEM((EPW,), jnp.int32),
            pltpu.VMEM((EPW,), jnp.int32),
            pltpu.VMEM((CG4, D), jnp.float32),
            pltpu.VMEM((CG4, D), jnp.float32),
            pltpu.VMEM((CG4, D), jnp.float32),
            pltpu.VMEM((CG4, D), jnp.float32),
            pltpu.SemaphoreType.DMA,
            pltpu.SemaphoreType.DMA,
        ],
    )
    return call(Pr, Pc, row, col)


CG6 = 40
NCH6 = EPW // CG6


def _scatter_body(eout_hbm, row_hbm, agg_hbm, ecnt_hbm,
                  ridx0, ridx1, ebuf0, ebuf1, obuf, zbuf, cbuf,
                  sh_agg, sh_cnt, sem0, sem1):
    core = lax.axis_index("c")
    sid = lax.axis_index("s")
    w = sid * NC + core
    base = w * EPW

    def zrow(r, _):
        for j in range(DEO // 16):
            zbuf[r, pl.ds(j * 16, 16)] = jnp.zeros((16,), jnp.float32)
        return 0

    lax.fori_loop(0, 128, zrow, 0)

    def fillc(r, _):
        cbuf[pl.ds(r * 16, 16)] = jnp.zeros((16,), jnp.float32)
        return 0

    lax.fori_loop(0, 8, fillc, 0)

    def zinit(t, _):
        sl = pl.ds(sid * TROWS + t * 128, 128)
        pltpu.sync_copy(zbuf, sh_agg.at[sl])
        pltpu.sync_copy(cbuf, sh_cnt.at[sl])
        return 0

    lax.fori_loop(0, TROWS // 128, zinit, 0)
    for o in (0, 16, CG6 - 16):
        obuf[pl.ds(o, 16)] = jnp.full((16,), 1.0, jnp.float32)
    plsc.subcore_barrier()

    bufs = ((ebuf0, ridx0, sem0), (ebuf1, ridx1, sem1))

    def issue(c, bset):
        ebuf, ridx, sem = bset
        o = base + c * CG6
        pltpu.async_copy(row_hbm.at[pl.ds(o, CG6)], ridx, sem)
        pltpu.async_copy(eout_hbm.at[pl.ds(o, CG6)], ebuf, sem)

    def finish(c, bset):
        ebuf, ridx, sem = bset
        o = base + c * CG6
        pltpu.make_async_copy(row_hbm.at[pl.ds(o, CG6)], ridx, sem).wait()
        pltpu.make_async_copy(eout_hbm.at[pl.ds(o, CG6)], ebuf, sem).wait()
        pltpu.sync_copy(ebuf, sh_agg.at[ridx], add=True)
        pltpu.sync_copy(obuf, sh_cnt.at[ridx], add=True)

    issue(0, bufs[0])

    def step(t2, _):
        for b in range(2):
            c = t2 * 2 + b

            @pl.when(c < NCH6)
            def _do():
                @pl.when(c + 1 < NCH6)
                def _nxt():
                    issue(c + 1, bufs[1 - b])

                finish(c, bufs[b])

        return 0

    lax.fori_loop(0, (NCH6 + 1) // 2, step, 0)
    plsc.subcore_barrier()

    def out(t, _):
        sl = pl.ds(sid * TROWS + t * 128, 128)
        pltpu.sync_copy(sh_agg.at[sl], zbuf)
        pltpu.sync_copy(zbuf, agg_hbm.at[core, sl])
        pltpu.sync_copy(sh_cnt.at[sl], cbuf)
        pltpu.sync_copy(cbuf, ecnt_hbm.at[core, sl])
        return 0

    lax.fori_loop(0, TROWS // 128, out, 0)


def _scatter_mean(eout, row):
    call = pl.kernel(
        _scatter_body,
        out_type=(jax.ShapeDtypeStruct((NC, NPAD, DEO), jnp.float32),
                  jax.ShapeDtypeStruct((NC, NPAD), jnp.float32)),
        mesh=_sc_mesh(),
        compiler_params=pltpu.CompilerParams(needs_layout_passes=False),
        scratch_types=[
            pltpu.VMEM((CG6,), jnp.int32),
            pltpu.VMEM((CG6,), jnp.int32),
            pltpu.VMEM((CG6, DEO), jnp.float32),
            pltpu.VMEM((CG6, DEO), jnp.float32),
            pltpu.VMEM((CG6,), jnp.float32),
            pltpu.VMEM((128, DEO), jnp.float32),
            pltpu.VMEM((128,), jnp.float32),
            pltpu.VMEM_SHARED((NPAD, DEO), jnp.float32),
            pltpu.VMEM_SHARED((NPAD,), jnp.float32),
            pltpu.SemaphoreType.DMA,
            pltpu.SemaphoreType.DMA,
        ],
    )
    A, C = call(eout, row)
    agg = A[0, :N] + A[1, :N]
    ecnt = C[0, :N] + C[1, :N]
    return agg / jnp.maximum(ecnt, 1.0)[:, None]


def kernel(x, x_src, edge_index, cg_edge_index, edge_attr, u, batch,
           batch_src, Wq, Wk, We1, be1, We2, be2, Wn1, bn1, Wn2, bn2,
           Wg1, bg1, Wg2, bg2):
    We1_r = We1[:D]
    We1_c = We1[D:2 * D]
    We1_e = We1[2 * D:2 * D + DE]
    We1_u = We1[2 * D + DE:]
    Wn1_x = Wn1[:D]
    Wn1_a = Wn1[D:D + DEO]
    Wn1_u = Wn1[D + DEO:]

    Wqp = jnp.pad(Wq, ((0, 0), (0, DAP - DA)))
    Wkp = jnp.pad(Wk, ((0, 0), (0, DAP - DA)))
    q, k = _proj(x, x_src, Wqp, Wkp)

    ci = cg_edge_index[0]
    cj = cg_edge_index[1]
    ssum, cnt = _attention_sums(q, k, ci, cj)
    a = ssum / jnp.maximum(cnt, 1.0)

    Wcat = jnp.concatenate([We1_r, We1_c, Wn1_x], axis=1)
    A = jnp.broadcast_to(a[:, None], (N, D))
    Un = (u @ We1_u + be1)[batch]
    Un2 = (u @ Wn1_u + bn1)[batch]
    Pr, Pc, Pn = _tables(x, Wcat, A, Un, Un2)

    row = edge_index[0]
    col = edge_index[1]
    G = _gather_sum(Pr, Pc, row, col)

    eout = _edge_mlp(G, edge_attr, We1_e, We2, be2.reshape(1, DEO))

    AggM = _scatter_mean(eout, row)

    Bh = jax.nn.one_hot(batch, B, dtype=jnp.float32)
    pcnt = jnp.maximum(jnp.sum(Bh, axis=0), 1.0)
    PcInv = jnp.broadcast_to((1.0 / pcnt)[:, None], (B, D))
    x_out, u_out = _node_global(Pn, AggM, Wn1_a, Wn2, bn2.reshape(1, D),
                                Bh, u, Wg1[:D], Wg1[D:], bg1.reshape(1, 128),
                                Wg2, bg2.reshape(1, DU), PcInv)
    return (x_out, eout, u_out)
